# Initial kernel scaffold; baseline (speedup 1.0000x reference)
import jax
import jax.numpy as jnp
from jax.experimental import pallas as pl


def kernel(x, h, c, edge_index, W_iou, U_iou, b_iou, U_f_w, U_f_b):
    raise NotImplementedError("write your pallas kernel here")


# speedup vs baseline: 6.4587x; 6.4587x over previous
import functools

import jax
import jax.numpy as jnp
from jax import lax
from jax.experimental import pallas as pl
from jax.experimental.pallas import tpu as pltpu
from jax.experimental.pallas import tpu_sc as plsc

NC = 2
NS = 16
CH = 128


def _pre_body(h_ref, c_ref, ufw_ref, ufb_ref, t_ref):
    h_blk = h_ref[...]
    hu = lax.dot_general(h_blk, ufw_ref[...], (((1,), (1,)), ((), ())),
                         preferred_element_type=jnp.float32)
    f = jax.nn.sigmoid(hu + ufb_ref[...])
    fc = f * c_ref[...]
    d = h_blk.shape[1]
    half = d // 2
    t_ref[0, :, 0:half] = h_blk[:, 0:half]
    t_ref[0, :, half:d] = fc[:, 0:half]
    t_ref[1, :, 0:half] = h_blk[:, half:d]
    t_ref[1, :, half:d] = fc[:, half:d]


def _pre(h, c, U_f_w, U_f_b, block_rows):
    n, d = h.shape
    grid = n // block_rows
    return pl.pallas_call(
        _pre_body,
        grid=(grid,),
        in_specs=[
            pl.BlockSpec((block_rows, d), lambda i: (i, 0)),
            pl.BlockSpec((block_rows, d), lambda i: (i, 0)),
            pl.BlockSpec((d, d), lambda i: (0, 0)),
            pl.BlockSpec((1, d), lambda i: (0, 0)),
        ],
        out_specs=pl.BlockSpec((NC, block_rows, d), lambda i: (0, i, 0)),
        out_shape=jax.ShapeDtypeStruct((NC, n, d), jnp.float32),
    )(h, c, U_f_w, U_f_b)


def _sc_segsum(t_flat, eidx, zeros, n, d, cpt, acc_rows):
    zrows = acc_rows // NS

    mesh = plsc.VectorSubcoreMesh(core_axis_name="c", subcore_axis_name="s")

    @functools.partial(
        pl.kernel,
        out_type=jax.ShapeDtypeStruct((NC, acc_rows, d), jnp.float32),
        mesh=mesh,
        scratch_types=[
            pltpu.VMEM_SHARED((acc_rows, d), jnp.float32),
            pltpu.VMEM((2, CH), jnp.int32),
            pltpu.VMEM((2, CH), jnp.int32),
            pltpu.VMEM((CH, d), jnp.float32),
            pltpu.VMEM((CH, d), jnp.float32),
            pltpu.SemaphoreType.DMA,
            pltpu.SemaphoreType.DMA,
        ],
    )
    def k(t_hbm, eidx_hbm, zeros_hbm, out_hbm,
          acc, idx0, idx1, rows0, rows1, sem0, sem1):
        core = lax.axis_index("c")
        sub = lax.axis_index("s")

        pltpu.sync_copy(zeros_hbm, acc.at[pl.ds(sub * zrows, zrows)])
        plsc.subcore_barrier()

        def start_gather(t, idx, buf, sem):
            pltpu.sync_copy(eidx_hbm.at[core, sub, t], idx)
            return pltpu.async_copy(t_hbm.at[idx.at[0]], buf, sem)

        def scatter(idx, buf):
            pltpu.sync_copy(buf, acc.at[idx.at[1]], add=True)

        @pl.loop(0, cpt // 2)
        def _(i):
            t = i * 2
            cp0 = start_gather(t, idx0, rows0, sem0)
            cp1 = start_gather(t + 1, idx1, rows1, sem1)
            cp0.wait()
            scatter(idx0, rows0)
            cp1.wait()
            scatter(idx1, rows1)

        if cpt % 2:
            start_gather(cpt - 1, idx0, rows0, sem0).wait()
            scatter(idx0, rows0)

        plsc.subcore_barrier()
        pltpu.sync_copy(acc.at[pl.ds(sub * zrows, zrows)],
                        out_hbm.at[core, pl.ds(sub * zrows, zrows)])

    return k(t_flat, eidx, zeros)


def _post_body(x_ref, parts_ref, wiou_ref, uiou_ref, biou_ref,
               h_new_ref, c_new_ref):
    d = x_ref.shape[1]
    half = d // 2
    ht = jnp.concatenate(
        [parts_ref[0, :, 0:half], parts_ref[1, :, 0:half]], axis=1)
    ca = jnp.concatenate(
        [parts_ref[0, :, half:d], parts_ref[1, :, half:d]], axis=1)
    iou = (
        lax.dot_general(x_ref[...], wiou_ref[...], (((1,), (1,)), ((), ())),
                        preferred_element_type=jnp.float32)
        + lax.dot_general(ht, uiou_ref[...], (((1,), (1,)), ((), ())),
                          preferred_element_type=jnp.float32)
        + biou_ref[...]
    )
    i_g = jax.nn.sigmoid(iou[:, 0:d])
    o_g = jax.nn.sigmoid(iou[:, d:2 * d])
    u_g = jnp.tanh(iou[:, 2 * d:3 * d])
    c_new = i_g * u_g + ca
    h_new = o_g * jnp.tanh(c_new)
    c_new_ref[...] = c_new
    h_new_ref[...] = h_new


def _post(x, parts, W_iou, U_iou, b_iou, block_rows):
    n, d = x.shape
    grid = n // block_rows
    return pl.pallas_call(
        _post_body,
        grid=(grid,),
        in_specs=[
            pl.BlockSpec((block_rows, d), lambda i: (i, 0)),
            pl.BlockSpec((NC, block_rows, d), lambda i: (0, i, 0)),
            pl.BlockSpec((3 * d, d), lambda i: (0, 0)),
            pl.BlockSpec((3 * d, d), lambda i: (0, 0)),
            pl.BlockSpec((1, 3 * d), lambda i: (0, 0)),
        ],
        out_specs=[
            pl.BlockSpec((block_rows, d), lambda i: (i, 0)),
            pl.BlockSpec((block_rows, d), lambda i: (i, 0)),
        ],
        out_shape=[
            jax.ShapeDtypeStruct((n, d), jnp.float32),
            jax.ShapeDtypeStruct((n, d), jnp.float32),
        ],
    )(x, parts, W_iou, U_iou, b_iou)


def kernel(x, h, c, edge_index, W_iou, U_iou, b_iou, U_f_w, U_f_b):
    n, d = h.shape
    e = edge_index.shape[1]

    src = edge_index[0].astype(jnp.int32)
    dst = edge_index[1].astype(jnp.int32)

    edges_per_tile_unit = NS * CH
    e_pad = -(-e // edges_per_tile_unit) * edges_per_tile_unit
    cpt = e_pad // edges_per_tile_unit
    pad = e_pad - e
    srcp = jnp.concatenate([src, jnp.zeros((pad,), jnp.int32)])
    dstp = jnp.concatenate([dst, jnp.full((pad,), n, jnp.int32)])

    acc_rows = -(-(n + 8) // (NS * 8)) * (NS * 8)
    zrows = acc_rows // NS

    sidx = jnp.stack([srcp, srcp + n]).reshape(NC, NS, cpt, 1, CH)
    didx = jnp.broadcast_to(dstp.reshape(1, NS, cpt, 1, CH),
                            (NC, NS, cpt, 1, CH))
    eidx = jnp.concatenate([sidx, didx], axis=3)
    zeros = jnp.zeros((zrows, d), jnp.float32)

    t_tab = _pre(h, c, U_f_w, U_f_b.reshape(1, d), block_rows=1000)
    parts = _sc_segsum(t_tab.reshape(NC * n, d), eidx, zeros,
                       n, d, cpt, acc_rows)
    h_new, c_new = _post(x, parts, W_iou, U_iou, b_iou, block_rows=1000)
    return (h_new, c_new)
